# baseline (speedup 1.0000x reference)
import functools

import jax
import jax.numpy as jnp
from jax import lax
from jax.experimental import pallas as pl
from jax.experimental.pallas import tpu as pltpu
from jax.experimental.pallas import tpu_sc as plsc

F = 26
B = 16384
FIELD = 40000
LANES = 16
NC, NS = 2, 16
NW = NC * NS
BPW = B // NW
N = F * BPW
GPW = BPW // LANES

_mesh = plsc.VectorSubcoreMesh(core_axis_name="c", subcore_axis_name="s")


@functools.partial(
    pl.kernel,
    mesh=_mesh,
    out_type=jax.ShapeDtypeStruct((B,), jnp.float32),
    scratch_types=[
        pltpu.VMEM((N,), jnp.int32),
        pltpu.VMEM((N,), jnp.float32),
        pltpu.VMEM((BPW,), jnp.float32),
        pltpu.VMEM((LANES,), jnp.float32),
        pltpu.SemaphoreType.DMA,
    ],
)
def _emb_sum(x_hbm, fc_hbm, bias_hbm, out_hbm, idx_v, vals_v, out_v, bias_v,
             sem):
    wid = lax.axis_index("s") * NC + lax.axis_index("c")

    xcp = pltpu.async_copy(x_hbm.at[pl.ds(wid * N, N)], idx_v, sem)
    pltpu.sync_copy(bias_hbm, bias_v)
    xcp.wait()

    def add_off(j, carry):
        sl = pl.ds(j * LANES, LANES)
        idx_v[sl] = idx_v[sl] + (j % F) * FIELD
        return carry

    lax.fori_loop(0, N // LANES, add_off, 0)

    pltpu.async_copy(fc_hbm.at[idx_v], vals_v, sem).wait()

    def reduce_group(g, carry):
        def body(f, acc):
            return acc + vals_v[pl.ds((g * F + f) * LANES, LANES)]

        out_v[pl.ds(g * LANES, LANES)] = lax.fori_loop(0, F, body, bias_v[...])
        return carry

    lax.fori_loop(0, GPW, reduce_group, 0)

    pltpu.sync_copy(out_v, out_hbm.at[pl.ds(wid * BPW, BPW)])


def kernel(x, fc_weight, bias):
    xg = x.reshape(B // LANES, LANES, F).transpose(0, 2, 1).reshape(-1)
    fc_flat = fc_weight.reshape(-1)
    bias_b = jnp.broadcast_to(bias.astype(jnp.float32), (LANES,))
    out = _emb_sum(xg, fc_flat, bias_b)
    return out.reshape(B, 1)


# speedup vs baseline: 1.3895x; 1.3895x over previous
import functools

import jax
import jax.numpy as jnp
from jax import lax
from jax.experimental import pallas as pl
from jax.experimental.pallas import tpu as pltpu
from jax.experimental.pallas import tpu_sc as plsc

F = 26
B = 16384
FIELD = 40000
LANES = 16
NC, NS = 2, 16
NW = NC * NS
BPW = B // NW
N = F * BPW

_mesh = plsc.VectorSubcoreMesh(core_axis_name="c", subcore_axis_name="s")


@functools.partial(
    pl.kernel,
    mesh=_mesh,
    out_type=jax.ShapeDtypeStruct((B,), jnp.float32),
    scratch_types=[
        pltpu.VMEM((N,), jnp.int32),
        pltpu.VMEM((N,), jnp.float32),
        pltpu.VMEM((BPW,), jnp.float32),
        pltpu.VMEM((LANES,), jnp.float32),
        pltpu.SemaphoreType.DMA,
    ],
)
def _emb_sum(xt_hbm, fc_hbm, bias_hbm, out_hbm, idx_v, vals_v, out_v, bias_v,
             sem):
    wid = lax.axis_index("s") * NC + lax.axis_index("c")
    base = wid * BPW

    copies = [
        pltpu.async_copy(
            xt_hbm.at[pl.ds(f * B + base, BPW)],
            idx_v.at[pl.ds(f * BPW, BPW)],
            sem,
        )
        for f in range(F)
    ]
    pltpu.sync_copy(bias_hbm, bias_v)
    for cp in copies:
        cp.wait()

    def add_off(j, carry):
        sl = pl.ds(j * LANES, LANES)
        idx_v[sl] = idx_v[sl] + (j // (BPW // LANES)) * FIELD
        return carry

    lax.fori_loop(0, N // LANES, add_off, 0)

    pltpu.async_copy(fc_hbm.at[idx_v], vals_v, sem).wait()

    def reduce_chunk(c, carry):
        def body(f, acc):
            return acc + vals_v[pl.ds(f * BPW + c * LANES, LANES)]

        out_v[pl.ds(c * LANES, LANES)] = lax.fori_loop(0, F, body, bias_v[...])
        return carry

    lax.fori_loop(0, BPW // LANES, reduce_chunk, 0)

    pltpu.sync_copy(out_v, out_hbm.at[pl.ds(base, BPW)])


def kernel(x, fc_weight, bias):
    xt = x.T.reshape(-1)
    fc_flat = fc_weight[:, 0]
    bias_b = jnp.broadcast_to(bias.astype(jnp.float32), (LANES,))
    out = _emb_sum(xt, fc_flat, bias_b)
    return out.reshape(B, 1)


# speedup vs baseline: 2.8318x; 2.0380x over previous
import functools

import jax
import jax.numpy as jnp
from jax import lax
from jax.experimental import pallas as pl
from jax.experimental.pallas import tpu as pltpu
from jax.experimental.pallas import tpu_sc as plsc

F = 26
B = 16384
FIELD = 40000
LANES = 16
NC, NS = 2, 16
NW = NC * NS
BPW = B // NW
N = F * BPW

_mesh = plsc.VectorSubcoreMesh(core_axis_name="c", subcore_axis_name="s")


@functools.partial(
    pl.kernel,
    mesh=_mesh,
    out_type=jax.ShapeDtypeStruct((B,), jnp.float32),
    scratch_types=[
        pltpu.VMEM((N,), jnp.int32),
        pltpu.VMEM((N,), jnp.float32),
        pltpu.VMEM((BPW,), jnp.float32),
        pltpu.VMEM((LANES,), jnp.float32),
        pltpu.SemaphoreType.DMA,
    ],
)
def _emb_sum(xt_hbm, fc_hbm, bias_hbm, out_hbm, idx_v, vals_v, out_v, bias_v,
             sem):
    wid = lax.axis_index("s") * NC + lax.axis_index("c")
    base = wid * BPW

    copies = [
        pltpu.async_copy(
            xt_hbm.at[f, pl.ds(base, BPW)],
            idx_v.at[pl.ds(f * BPW, BPW)],
            sem,
        )
        for f in range(F)
    ]
    pltpu.sync_copy(bias_hbm, bias_v)
    for cp in copies:
        cp.wait()

    def add_off(j, carry):
        sl = pl.ds(j * LANES, LANES)
        idx_v[sl] = idx_v[sl] + (j // (BPW // LANES)) * FIELD
        return carry

    lax.fori_loop(0, N // LANES, add_off, 0)

    pltpu.async_copy(fc_hbm.at[0].at[idx_v], vals_v, sem).wait()

    def reduce_chunk(c, carry):
        def body(f, acc):
            return acc + vals_v[pl.ds(f * BPW + c * LANES, LANES)]

        out_v[pl.ds(c * LANES, LANES)] = lax.fori_loop(0, F, body, bias_v[...])
        return carry

    lax.fori_loop(0, BPW // LANES, reduce_chunk, 0)

    pltpu.sync_copy(out_v, out_hbm.at[pl.ds(base, BPW)])


def kernel(x, fc_weight, bias):
    xt = x.T
    fc_t = fc_weight.T
    bias_b = jnp.broadcast_to(bias.astype(jnp.float32), (LANES,))
    out = _emb_sum(xt, fc_t, bias_b)
    return out.reshape(B, 1)


# speedup vs baseline: 3.0416x; 1.0741x over previous
import functools

import jax
import jax.numpy as jnp
from jax import lax
from jax.experimental import pallas as pl
from jax.experimental.pallas import tpu as pltpu
from jax.experimental.pallas import tpu_sc as plsc

F = 26
B = 16384
FIELD = 40000
LANES = 16
NC, NS = 2, 16
NW = NC * NS
BPW = B // NW
N = F * BPW

_mesh = plsc.VectorSubcoreMesh(core_axis_name="c", subcore_axis_name="s")


@functools.partial(
    pl.kernel,
    mesh=_mesh,
    out_type=jax.ShapeDtypeStruct((B,), jnp.float32),
    scratch_types=[
        pltpu.VMEM((N,), jnp.int32),
        pltpu.VMEM((N,), jnp.float32),
        pltpu.VMEM((BPW,), jnp.float32),
        pltpu.VMEM((LANES,), jnp.float32),
        pltpu.SemaphoreType.DMA,
    ],
)
def _emb_sum(xt_hbm, fc_hbm, bias_hbm, out_hbm, idx_v, vals_v, out_v, bias_v,
             sem):
    wid = lax.axis_index("s") * NC + lax.axis_index("c")
    base = wid * BPW

    copies = [
        pltpu.async_copy(
            xt_hbm.at[f, pl.ds(base, BPW)],
            idx_v.at[pl.ds(f * BPW, BPW)],
            sem,
        )
        for f in range(F)
    ]
    pltpu.sync_copy(bias_hbm, bias_v)
    for cp in copies:
        cp.wait()

    def add_off(c, carry):
        for f in range(F):
            sl = pl.ds(f * BPW + c * LANES, LANES)
            idx_v[sl] = idx_v[sl] + f * FIELD
        return carry

    lax.fori_loop(0, BPW // LANES, add_off, 0)

    pltpu.async_copy(fc_hbm.at[0].at[idx_v], vals_v, sem).wait()

    def reduce_chunk(c, carry):
        acc = bias_v[...]
        for f in range(F):
            acc = acc + vals_v[pl.ds(f * BPW + c * LANES, LANES)]
        out_v[pl.ds(c * LANES, LANES)] = acc
        return carry

    lax.fori_loop(0, BPW // LANES, reduce_chunk, 0)

    pltpu.sync_copy(out_v, out_hbm.at[pl.ds(base, BPW)])


def kernel(x, fc_weight, bias):
    xt = x.T
    fc_t = fc_weight.T
    bias_b = jnp.broadcast_to(bias.astype(jnp.float32), (LANES,))
    out = _emb_sum(xt, fc_t, bias_b)
    return out.reshape(B, 1)
